# baseline (speedup 1.0000x reference)
import functools

import jax
import jax.numpy as jnp
from jax import lax
from jax.experimental import pallas as pl
from jax.experimental.pallas import tpu as pltpu
from jax.experimental.pallas import tpu_sc as plsc

N_NODES = 10000
N_EDGES = 320000
DIM = 128
N_GRAPHS = 16
OUT_CH = 10

NC = 2
NS = 16
NW = NC * NS
CHUNK = 128
NCHUNK = 80
EPW = CHUNK * NCHUNK
N_EPAD = NW * EPW
TRASH = N_NODES
N_ACC = N_NODES + 16
RPT = 624
TAIL = N_NODES - NS * RPT
ZROWS = 208

_LANES = 16


def _agg_body(h_hbm, sd_hbm, out_hbm,
              acc, idx0, idx1, rows0, rows1, sem0, sem1):
    c = lax.axis_index("c")
    s = lax.axis_index("s")
    wid = c * NS + s

    @pl.loop(0, CHUNK)
    def _zero(r):
        for j in range(0, DIM, _LANES):
            rows0[r, pl.ds(j, _LANES)] = jnp.zeros((_LANES,), jnp.float32)

    @pl.loop(0, RPT // CHUNK)
    def _init(j):
        pltpu.sync_copy(rows0, acc.at[pl.ds(s * RPT + j * CHUNK, CHUNK)])

    pltpu.sync_copy(rows0.at[pl.ds(0, RPT % CHUNK)],
                    acc.at[pl.ds(s * RPT + RPT - RPT % CHUNK, RPT % CHUNK)])

    @pl.when(s == NS - 1)
    def _init_tail():
        pltpu.sync_copy(rows0.at[pl.ds(0, N_ACC - NS * RPT)],
                        acc.at[pl.ds(NS * RPT, N_ACC - NS * RPT)])

    plsc.subcore_barrier()

    @pl.loop(0, NCHUNK)
    def _chunk(t):
        pltpu.sync_copy(sd_hbm.at[wid].at[t].at[0], idx0)
        pltpu.sync_copy(sd_hbm.at[wid].at[t].at[1], idx1)
        pltpu.async_copy(h_hbm.at[idx0], rows0, sem0).wait()
        pltpu.sync_copy(rows0, acc.at[idx1], add=True)

    plsc.subcore_barrier()

    pltpu.sync_copy(acc.at[pl.ds(s * RPT, RPT)],
                    out_hbm.at[c].at[pl.ds(s * RPT, RPT)])

    @pl.when(s == NS - 1)
    def _out_tail():
        pltpu.sync_copy(acc.at[pl.ds(NS * RPT, TAIL)],
                        out_hbm.at[c].at[pl.ds(NS * RPT, TAIL)])


_agg = pl.kernel(
    _agg_body,
    out_type=jax.ShapeDtypeStruct((NC, N_NODES, DIM), jnp.float32),
    mesh=plsc.VectorSubcoreMesh(core_axis_name="c", subcore_axis_name="s"),
    scratch_types=[
        pltpu.VMEM_SHARED((N_ACC, DIM), jnp.float32),
        pltpu.VMEM((CHUNK,), jnp.int32),
        pltpu.VMEM((CHUNK,), jnp.int32),
        pltpu.VMEM((CHUNK, DIM), jnp.float32),
        pltpu.VMEM((CHUNK, DIM), jnp.float32),
        pltpu.SemaphoreType.DMA,
        pltpu.SemaphoreType.DMA,
    ],
)


_HI = lax.Precision.HIGHEST
_RB = 1000
_NRB = N_NODES // _RB


def _layer_math(x_blk, a_ref, w1s_ref, b1s_ref, w2_ref, b2_ref):
    h = x_blk + a_ref[0] + a_ref[1]
    t = jnp.dot(h, w1s_ref[...], precision=_HI) + b1s_ref[...]
    t = jnp.maximum(t, 0.0)
    o = jnp.dot(t, w2_ref[...], precision=_HI) + b2_ref[...]
    return jnp.maximum(o, 0.0)


def _mlp_body(x_ref, a_ref, w1s_ref, b1s_ref, w2_ref, b2_ref, o_ref):
    o_ref[...] = _layer_math(x_ref[...], a_ref, w1s_ref, b1s_ref,
                             w2_ref, b2_ref)


def _mlp(h, agg, w1s, b1s, w2, b2):
    return pl.pallas_call(
        _mlp_body,
        grid=(_NRB,),
        in_specs=[
            pl.BlockSpec((_RB, DIM), lambda i: (i, 0)),
            pl.BlockSpec((NC, _RB, DIM), lambda i: (0, i, 0)),
            pl.BlockSpec((DIM, DIM), lambda i: (0, 0)),
            pl.BlockSpec((1, DIM), lambda i: (0, 0)),
            pl.BlockSpec((DIM, DIM), lambda i: (0, 0)),
            pl.BlockSpec((1, DIM), lambda i: (0, 0)),
        ],
        out_specs=pl.BlockSpec((_RB, DIM), lambda i: (i, 0)),
        out_shape=jax.ShapeDtypeStruct((N_NODES, DIM), jnp.float32),
    )(h, agg, w1s, b1s, w2, b2)


def _head_body(x_ref, a_ref, batch_ref, w1s_ref, b1s_ref, w2_ref, b2_ref,
               l1w_ref, l1b_ref, l2w_ref, l2b_ref, o_ref, pool_acc):
    i = pl.program_id(0)
    h5 = _layer_math(x_ref[...], a_ref, w1s_ref, b1s_ref, w2_ref, b2_ref)
    b = batch_ref[0, 0, :]
    onehot = (b[:, None] == lax.broadcasted_iota(
        jnp.int32, (1, N_GRAPHS), 1)).astype(jnp.float32)
    part = lax.dot_general(onehot, h5, (((0,), (0,)), ((), ())),
                           precision=_HI)

    @pl.when(i == 0)
    def _first():
        pool_acc[...] = part

    @pl.when(i > 0)
    def _rest():
        pool_acc[...] += part

    @pl.when(i == _NRB - 1)
    def _final():
        pooled = pool_acc[...]
        u = jnp.dot(pooled, l1w_ref[...], precision=_HI) + l1b_ref[...]
        u = jnp.maximum(u, 0.0)
        o = jnp.dot(u, l2w_ref[...], precision=_HI) + l2b_ref[...]
        m = jnp.max(o, axis=-1, keepdims=True)
        e = o - m
        o_ref[...] = e - jnp.log(jnp.sum(jnp.exp(e), axis=-1, keepdims=True))


def _head(h, agg, batch_r, w1s, b1s, w2, b2, l1w, l1b, l2w, l2b):
    return pl.pallas_call(
        _head_body,
        grid=(_NRB,),
        in_specs=[
            pl.BlockSpec((_RB, DIM), lambda i: (i, 0)),
            pl.BlockSpec((NC, _RB, DIM), lambda i: (0, i, 0)),
            pl.BlockSpec((1, 1, _RB), lambda i: (i, 0, 0)),
            pl.BlockSpec((DIM, DIM), lambda i: (0, 0)),
            pl.BlockSpec((1, DIM), lambda i: (0, 0)),
            pl.BlockSpec((DIM, DIM), lambda i: (0, 0)),
            pl.BlockSpec((1, DIM), lambda i: (0, 0)),
            pl.BlockSpec((DIM, DIM), lambda i: (0, 0)),
            pl.BlockSpec((1, DIM), lambda i: (0, 0)),
            pl.BlockSpec((DIM, OUT_CH), lambda i: (0, 0)),
            pl.BlockSpec((1, OUT_CH), lambda i: (0, 0)),
        ],
        out_specs=pl.BlockSpec((N_GRAPHS, OUT_CH), lambda i: (0, 0)),
        out_shape=jax.ShapeDtypeStruct((N_GRAPHS, OUT_CH), jnp.float32),
        scratch_shapes=[pltpu.VMEM((N_GRAPHS, DIM), jnp.float32)],
    )(h, agg, batch_r, w1s, b1s, w2, b2, l1w, l1b, l2w, l2b)


def kernel(x, edge_index, batch, params):
    npad = N_EPAD - N_EDGES
    src = jnp.concatenate(
        [edge_index[0].astype(jnp.int32),
         jnp.zeros((npad,), jnp.int32)]).reshape(NW, NCHUNK, CHUNK)
    dst = jnp.concatenate(
        [edge_index[1].astype(jnp.int32),
         jnp.full((npad,), TRASH, jnp.int32)]).reshape(NW, NCHUNK, CHUNK)
    sd = jnp.stack([src, dst], axis=2)
    batch_r = batch.astype(jnp.int32).reshape(_NRB, 1, _RB)

    bn_rsqrt = 1.0 / jnp.sqrt(jnp.float32(1.0 + 1e-5))
    h = x
    out = None
    for li in range(1, 6):
        p = params[f"conv{li}"]
        scale = p["g"] * bn_rsqrt
        w1s = p["W1"] * scale[None, :]
        b1s = (p["b1"] * scale + p["b"]).reshape(1, DIM)
        w2 = p["W2"]
        b2 = p["b2"].reshape(1, DIM)
        agg = _agg(h, sd)
        if li < 5:
            h = _mlp(h, agg, w1s, b1s, w2, b2)
        else:
            out = _head(h, agg, batch_r, w1s, b1s, w2, b2,
                        params["lin1_W"], params["lin1_b"].reshape(1, DIM),
                        params["lin2_W"], params["lin2_b"].reshape(1, OUT_CH))
    return out


# speedup vs baseline: 1.2139x; 1.2139x over previous
import functools

import jax
import jax.numpy as jnp
from jax import lax
from jax.experimental import pallas as pl
from jax.experimental.pallas import tpu as pltpu
from jax.experimental.pallas import tpu_sc as plsc

N_NODES = 10000
N_EDGES = 320000
DIM = 128
N_GRAPHS = 16
OUT_CH = 10

NC = 2
NS = 16
NW = NC * NS
CHUNK = 128
NCHUNK = 80
EPW = CHUNK * NCHUNK
N_EPAD = NW * EPW
TRASH = N_NODES
N_ACC = N_NODES + 16
RPT = 624
TAIL = N_NODES - NS * RPT
ZROWS = 208

_LANES = 16


def _agg_body(h_hbm, sd_hbm, out_hbm,
              acc, idx0, idx1, rows0, rows1, sem0, sem1):
    c = lax.axis_index("c")
    s = lax.axis_index("s")
    wid = c * NS + s

    @pl.loop(0, CHUNK)
    def _zero(r):
        for j in range(0, DIM, _LANES):
            rows0[r, pl.ds(j, _LANES)] = jnp.zeros((_LANES,), jnp.float32)

    @pl.loop(0, RPT // CHUNK)
    def _init(j):
        pltpu.sync_copy(rows0, acc.at[pl.ds(s * RPT + j * CHUNK, CHUNK)])

    pltpu.sync_copy(rows0.at[pl.ds(0, RPT % CHUNK)],
                    acc.at[pl.ds(s * RPT + RPT - RPT % CHUNK, RPT % CHUNK)])

    @pl.when(s == NS - 1)
    def _init_tail():
        pltpu.sync_copy(rows0.at[pl.ds(0, N_ACC - NS * RPT)],
                        acc.at[pl.ds(NS * RPT, N_ACC - NS * RPT)])

    plsc.subcore_barrier()

    @pl.loop(0, NCHUNK)
    def _chunk(t):
        pltpu.sync_copy(sd_hbm.at[wid].at[t].at[0], idx0)
        pltpu.sync_copy(sd_hbm.at[wid].at[t].at[1], idx1)
        pltpu.async_copy(h_hbm.at[idx0], rows0, sem0).wait()
        pltpu.sync_copy(rows0, acc.at[idx1], add=True)

    plsc.subcore_barrier()

    pltpu.sync_copy(acc.at[pl.ds(s * RPT, RPT)],
                    out_hbm.at[c].at[pl.ds(s * RPT, RPT)])

    @pl.when(s == NS - 1)
    def _out_tail():
        pltpu.sync_copy(acc.at[pl.ds(NS * RPT, TAIL)],
                        out_hbm.at[c].at[pl.ds(NS * RPT, TAIL)])


_agg = pl.kernel(
    _agg_body,
    out_type=jax.ShapeDtypeStruct((NC, N_NODES, DIM), jnp.float32),
    mesh=plsc.VectorSubcoreMesh(core_axis_name="c", subcore_axis_name="s"),
    scratch_types=[
        pltpu.VMEM_SHARED((N_ACC, DIM), jnp.float32),
        pltpu.VMEM((CHUNK,), jnp.int32),
        pltpu.VMEM((CHUNK,), jnp.int32),
        pltpu.VMEM((CHUNK, DIM), jnp.float32),
        pltpu.VMEM((CHUNK, DIM), jnp.float32),
        pltpu.SemaphoreType.DMA,
        pltpu.SemaphoreType.DMA,
    ],
)


_HI = lax.Precision.HIGHEST
_RB = 1000
_NRB = N_NODES // _RB


def _layer_math(x_blk, a_ref, w1s_ref, b1s_ref, w2_ref, b2_ref):
    h = x_blk + a_ref[0] + a_ref[1]
    t = jnp.dot(h, w1s_ref[...], precision=_HI) + b1s_ref[...]
    t = jnp.maximum(t, 0.0)
    o = jnp.dot(t, w2_ref[...], precision=_HI) + b2_ref[...]
    return jnp.maximum(o, 0.0)


def _mlp_body(x_ref, a_ref, w1s_ref, b1s_ref, w2_ref, b2_ref, o_ref):
    o_ref[...] = _layer_math(x_ref[...], a_ref, w1s_ref, b1s_ref,
                             w2_ref, b2_ref)


def _mlp(h, agg, w1s, b1s, w2, b2):
    return pl.pallas_call(
        _mlp_body,
        grid=(_NRB,),
        in_specs=[
            pl.BlockSpec((_RB, DIM), lambda i: (i, 0)),
            pl.BlockSpec((NC, _RB, DIM), lambda i: (0, i, 0)),
            pl.BlockSpec((DIM, DIM), lambda i: (0, 0)),
            pl.BlockSpec((1, DIM), lambda i: (0, 0)),
            pl.BlockSpec((DIM, DIM), lambda i: (0, 0)),
            pl.BlockSpec((1, DIM), lambda i: (0, 0)),
        ],
        out_specs=pl.BlockSpec((_RB, DIM), lambda i: (i, 0)),
        out_shape=jax.ShapeDtypeStruct((N_NODES, DIM), jnp.float32),
    )(h, agg, w1s, b1s, w2, b2)


def _head_body(x_ref, a_ref, batch_ref, w1s_ref, b1s_ref, w2_ref, b2_ref,
               l1w_ref, l1b_ref, l2w_ref, l2b_ref, o_ref, pool_acc):
    i = pl.program_id(0)
    h5 = _layer_math(x_ref[...], a_ref, w1s_ref, b1s_ref, w2_ref, b2_ref)
    b = batch_ref[0, 0, :]
    onehot = (b[:, None] == lax.broadcasted_iota(
        jnp.int32, (1, N_GRAPHS), 1)).astype(jnp.float32)
    part = lax.dot_general(onehot, h5, (((0,), (0,)), ((), ())),
                           precision=_HI)

    @pl.when(i == 0)
    def _first():
        pool_acc[...] = part

    @pl.when(i > 0)
    def _rest():
        pool_acc[...] += part

    @pl.when(i == _NRB - 1)
    def _final():
        pooled = pool_acc[...]
        u = jnp.dot(pooled, l1w_ref[...], precision=_HI) + l1b_ref[...]
        u = jnp.maximum(u, 0.0)
        o = jnp.dot(u, l2w_ref[...], precision=_HI) + l2b_ref[...]
        m = jnp.max(o, axis=-1, keepdims=True)
        e = o - m
        o_ref[...] = e - jnp.log(jnp.sum(jnp.exp(e), axis=-1, keepdims=True))


def _head(h, agg, batch_r, w1s, b1s, w2, b2, l1w, l1b, l2w, l2b):
    return pl.pallas_call(
        _head_body,
        grid=(_NRB,),
        in_specs=[
            pl.BlockSpec((_RB, DIM), lambda i: (i, 0)),
            pl.BlockSpec((NC, _RB, DIM), lambda i: (0, i, 0)),
            pl.BlockSpec((1, 1, _RB), lambda i: (i, 0, 0)),
            pl.BlockSpec((DIM, DIM), lambda i: (0, 0)),
            pl.BlockSpec((1, DIM), lambda i: (0, 0)),
            pl.BlockSpec((DIM, DIM), lambda i: (0, 0)),
            pl.BlockSpec((1, DIM), lambda i: (0, 0)),
            pl.BlockSpec((DIM, DIM), lambda i: (0, 0)),
            pl.BlockSpec((1, DIM), lambda i: (0, 0)),
            pl.BlockSpec((DIM, OUT_CH), lambda i: (0, 0)),
            pl.BlockSpec((1, OUT_CH), lambda i: (0, 0)),
        ],
        out_specs=pl.BlockSpec((N_GRAPHS, OUT_CH), lambda i: (0, 0)),
        out_shape=jax.ShapeDtypeStruct((N_GRAPHS, OUT_CH), jnp.float32),
        scratch_shapes=[pltpu.VMEM((N_GRAPHS, DIM), jnp.float32)],
    )(h, agg, batch_r, w1s, b1s, w2, b2, l1w, l1b, l2w, l2b)


def kernel(x, edge_index, batch, params):
    ppw = EPW - N_EDGES // NW
    pad_dst = jnp.broadcast_to(
        TRASH + (jnp.arange(ppw, dtype=jnp.int32) % 16), (NW, ppw))
    src = jnp.concatenate(
        [edge_index[0].astype(jnp.int32).reshape(NW, N_EDGES // NW),
         jnp.zeros((NW, ppw), jnp.int32)], axis=1).reshape(NW, NCHUNK, CHUNK)
    dst = jnp.concatenate(
        [edge_index[1].astype(jnp.int32).reshape(NW, N_EDGES // NW),
         pad_dst], axis=1).reshape(NW, NCHUNK, CHUNK)
    sd = jnp.stack([src, dst], axis=2)
    batch_r = batch.astype(jnp.int32).reshape(_NRB, 1, _RB)

    bn_rsqrt = 1.0 / jnp.sqrt(jnp.float32(1.0 + 1e-5))
    h = x
    out = None
    for li in range(1, 6):
        p = params[f"conv{li}"]
        scale = p["g"] * bn_rsqrt
        w1s = p["W1"] * scale[None, :]
        b1s = (p["b1"] * scale + p["b"]).reshape(1, DIM)
        w2 = p["W2"]
        b2 = p["b2"].reshape(1, DIM)
        agg = _agg(h, sd)
        if li < 5:
            h = _mlp(h, agg, w1s, b1s, w2, b2)
        else:
            out = _head(h, agg, batch_r, w1s, b1s, w2, b2,
                        params["lin1_W"], params["lin1_b"].reshape(1, DIM),
                        params["lin2_W"], params["lin2_b"].reshape(1, OUT_CH))
    return out


# speedup vs baseline: 2.8026x; 2.3088x over previous
import functools

import jax
import jax.numpy as jnp
from jax import lax
from jax.experimental import pallas as pl
from jax.experimental.pallas import tpu as pltpu
from jax.experimental.pallas import tpu_sc as plsc

N_NODES = 10000
N_EDGES = 320000
DIM = 128
N_GRAPHS = 16
OUT_CH = 10

NC = 2
NS = 16
NW = NC * NS
CHUNK = 80
EPW = N_EDGES // NW
NCHUNK = EPW // CHUNK
RPT = 624
TAIL = N_NODES - NS * RPT

_LANES = 16


def _agg_body(h_hbm, src_hbm, dst_hbm, out_hbm,
              acc, src_v, dst_v, rows0, sem0):
    c = lax.axis_index("c")
    s = lax.axis_index("s")
    wid = c * NS + s

    src_cp = pltpu.async_copy(src_hbm.at[pl.ds(wid * EPW, EPW)], src_v, sem0)
    dst_cp = pltpu.async_copy(dst_hbm.at[wid], dst_v, sem0)

    @pl.loop(0, CHUNK)
    def _zero(r):
        for j in range(0, DIM, _LANES):
            rows0[r, pl.ds(j, _LANES)] = jnp.zeros((_LANES,), jnp.float32)

    @pl.loop(0, RPT // CHUNK)
    def _init(j):
        pltpu.sync_copy(rows0, acc.at[pl.ds(s * RPT + j * CHUNK, CHUNK)])

    pltpu.sync_copy(rows0.at[pl.ds(0, RPT % CHUNK)],
                    acc.at[pl.ds(s * RPT + RPT - RPT % CHUNK, RPT % CHUNK)])

    @pl.when(s == NS - 1)
    def _init_tail():
        pltpu.sync_copy(rows0.at[pl.ds(0, TAIL)],
                        acc.at[pl.ds(NS * RPT, TAIL)])

    src_cp.wait()
    dst_cp.wait()
    plsc.subcore_barrier()

    @pl.loop(0, NCHUNK)
    def _chunk(t):
        pltpu.async_copy(
            h_hbm.at[src_v.at[pl.ds(t * CHUNK, CHUNK)]], rows0, sem0).wait()
        pltpu.sync_copy(rows0, acc.at[dst_v.at[t]], add=True)

    plsc.subcore_barrier()

    pltpu.sync_copy(acc.at[pl.ds(s * RPT, RPT)],
                    out_hbm.at[c].at[pl.ds(s * RPT, RPT)])

    @pl.when(s == NS - 1)
    def _out_tail():
        pltpu.sync_copy(acc.at[pl.ds(NS * RPT, TAIL)],
                        out_hbm.at[c].at[pl.ds(NS * RPT, TAIL)])


_agg = pl.kernel(
    _agg_body,
    out_type=jax.ShapeDtypeStruct((NC, N_NODES, DIM), jnp.float32),
    mesh=plsc.VectorSubcoreMesh(core_axis_name="c", subcore_axis_name="s"),
    scratch_types=[
        pltpu.VMEM_SHARED((N_NODES, DIM), jnp.float32),
        pltpu.VMEM((EPW,), jnp.int32),
        pltpu.VMEM((NCHUNK, CHUNK), jnp.int32),
        pltpu.VMEM((CHUNK, DIM), jnp.float32),
        pltpu.SemaphoreType.DMA,
    ],
)


_HI = lax.Precision.HIGHEST
_RB = 1000
_NRB = N_NODES // _RB


def _layer_math(x_blk, a_ref, w1s_ref, b1s_ref, w2_ref, b2_ref):
    h = x_blk + a_ref[0] + a_ref[1]
    t = jnp.dot(h, w1s_ref[...], precision=_HI) + b1s_ref[...]
    t = jnp.maximum(t, 0.0)
    o = jnp.dot(t, w2_ref[...], precision=_HI) + b2_ref[...]
    return jnp.maximum(o, 0.0)


def _mlp_body(x_ref, a_ref, w1s_ref, b1s_ref, w2_ref, b2_ref, o_ref):
    o_ref[...] = _layer_math(x_ref[...], a_ref, w1s_ref, b1s_ref,
                             w2_ref, b2_ref)


def _mlp(h, agg, w1s, b1s, w2, b2):
    return pl.pallas_call(
        _mlp_body,
        grid=(_NRB,),
        in_specs=[
            pl.BlockSpec((_RB, DIM), lambda i: (i, 0)),
            pl.BlockSpec((NC, _RB, DIM), lambda i: (0, i, 0)),
            pl.BlockSpec((DIM, DIM), lambda i: (0, 0)),
            pl.BlockSpec((1, DIM), lambda i: (0, 0)),
            pl.BlockSpec((DIM, DIM), lambda i: (0, 0)),
            pl.BlockSpec((1, DIM), lambda i: (0, 0)),
        ],
        out_specs=pl.BlockSpec((_RB, DIM), lambda i: (i, 0)),
        out_shape=jax.ShapeDtypeStruct((N_NODES, DIM), jnp.float32),
    )(h, agg, w1s, b1s, w2, b2)


def _head_body(x_ref, a_ref, batch_ref, w1s_ref, b1s_ref, w2_ref, b2_ref,
               l1w_ref, l1b_ref, l2w_ref, l2b_ref, o_ref, pool_acc):
    i = pl.program_id(0)
    h5 = _layer_math(x_ref[...], a_ref, w1s_ref, b1s_ref, w2_ref, b2_ref)
    b = batch_ref[0, 0, :]
    onehot = (b[:, None] == lax.broadcasted_iota(
        jnp.int32, (1, N_GRAPHS), 1)).astype(jnp.float32)
    part = lax.dot_general(onehot, h5, (((0,), (0,)), ((), ())),
                           precision=_HI)

    @pl.when(i == 0)
    def _first():
        pool_acc[...] = part

    @pl.when(i > 0)
    def _rest():
        pool_acc[...] += part

    @pl.when(i == _NRB - 1)
    def _final():
        pooled = pool_acc[...]
        u = jnp.dot(pooled, l1w_ref[...], precision=_HI) + l1b_ref[...]
        u = jnp.maximum(u, 0.0)
        o = jnp.dot(u, l2w_ref[...], precision=_HI) + l2b_ref[...]
        m = jnp.max(o, axis=-1, keepdims=True)
        e = o - m
        o_ref[...] = e - jnp.log(jnp.sum(jnp.exp(e), axis=-1, keepdims=True))


def _head(h, agg, batch_r, w1s, b1s, w2, b2, l1w, l1b, l2w, l2b):
    return pl.pallas_call(
        _head_body,
        grid=(_NRB,),
        in_specs=[
            pl.BlockSpec((_RB, DIM), lambda i: (i, 0)),
            pl.BlockSpec((NC, _RB, DIM), lambda i: (0, i, 0)),
            pl.BlockSpec((1, 1, _RB), lambda i: (i, 0, 0)),
            pl.BlockSpec((DIM, DIM), lambda i: (0, 0)),
            pl.BlockSpec((1, DIM), lambda i: (0, 0)),
            pl.BlockSpec((DIM, DIM), lambda i: (0, 0)),
            pl.BlockSpec((1, DIM), lambda i: (0, 0)),
            pl.BlockSpec((DIM, DIM), lambda i: (0, 0)),
            pl.BlockSpec((1, DIM), lambda i: (0, 0)),
            pl.BlockSpec((DIM, OUT_CH), lambda i: (0, 0)),
            pl.BlockSpec((1, OUT_CH), lambda i: (0, 0)),
        ],
        out_specs=pl.BlockSpec((N_GRAPHS, OUT_CH), lambda i: (0, 0)),
        out_shape=jax.ShapeDtypeStruct((N_GRAPHS, OUT_CH), jnp.float32),
        scratch_shapes=[pltpu.VMEM((N_GRAPHS, DIM), jnp.float32)],
    )(h, agg, batch_r, w1s, b1s, w2, b2, l1w, l1b, l2w, l2b)


def kernel(x, edge_index, batch, params):
    src = edge_index[0].astype(jnp.int32)
    dst = edge_index[1].astype(jnp.int32).reshape(NW, NCHUNK, CHUNK)
    batch_r = batch.astype(jnp.int32).reshape(_NRB, 1, _RB)

    bn_rsqrt = 1.0 / jnp.sqrt(jnp.float32(1.0 + 1e-5))
    h = x
    out = None
    for li in range(1, 6):
        p = params[f"conv{li}"]
        scale = p["g"] * bn_rsqrt
        w1s = p["W1"] * scale[None, :]
        b1s = (p["b1"] * scale + p["b"]).reshape(1, DIM)
        w2 = p["W2"]
        b2 = p["b2"].reshape(1, DIM)
        agg = _agg(h, src, dst)
        if li < 5:
            h = _mlp(h, agg, w1s, b1s, w2, b2)
        else:
            out = _head(h, agg, batch_r, w1s, b1s, w2, b2,
                        params["lin1_W"], params["lin1_b"].reshape(1, DIM),
                        params["lin2_W"], params["lin2_b"].reshape(1, OUT_CH))
    return out


# speedup vs baseline: 4.3914x; 1.5669x over previous
import functools

import jax
import jax.numpy as jnp
from jax import lax
from jax.experimental import pallas as pl
from jax.experimental.pallas import tpu as pltpu
from jax.experimental.pallas import tpu_sc as plsc

N_NODES = 10000
N_EDGES = 320000
DIM = 128
N_GRAPHS = 16
OUT_CH = 10

NC = 2
NS = 16
NW = NC * NS
CHUNK = 80
EPW = N_EDGES // NW
NCHUNK = EPW // CHUNK
RPT = 624
TAIL = N_NODES - NS * RPT

_LANES = 16


def _agg_body(h_hbm, src_hbm, dst_hbm, out_hbm,
              acc, src_v, dst_v, rows0, rows1, sem0, sem1):
    c = lax.axis_index("c")
    s = lax.axis_index("s")
    wid = c * NS + s

    src_cp = pltpu.async_copy(src_hbm.at[pl.ds(wid * EPW, EPW)], src_v, sem0)
    dst_cp = pltpu.async_copy(dst_hbm.at[wid], dst_v, sem0)

    @pl.loop(0, CHUNK)
    def _zero(r):
        for j in range(0, DIM, _LANES):
            rows0[r, pl.ds(j, _LANES)] = jnp.zeros((_LANES,), jnp.float32)

    @pl.loop(0, RPT // CHUNK)
    def _init(j):
        pltpu.sync_copy(rows0, acc.at[pl.ds(s * RPT + j * CHUNK, CHUNK)])

    pltpu.sync_copy(rows0.at[pl.ds(0, RPT % CHUNK)],
                    acc.at[pl.ds(s * RPT + RPT - RPT % CHUNK, RPT % CHUNK)])

    @pl.when(s == NS - 1)
    def _init_tail():
        pltpu.sync_copy(rows0.at[pl.ds(0, TAIL)],
                        acc.at[pl.ds(NS * RPT, TAIL)])

    src_cp.wait()
    dst_cp.wait()
    plsc.subcore_barrier()

    def _start(t, buf, sem):
        pltpu.async_copy(h_hbm.at[src_v.at[pl.ds(t * CHUNK, CHUNK)]], buf, sem)

    def _finish(t, buf, sem):
        pltpu.make_async_copy(
            h_hbm.at[src_v.at[pl.ds(t * CHUNK, CHUNK)]], buf, sem).wait()
        pltpu.sync_copy(buf, acc.at[dst_v.at[t]], add=True)

    _start(0, rows0, sem0)

    @pl.loop(0, NCHUNK // 2)
    def _chunk(k):
        t0 = 2 * k
        _start(t0 + 1, rows1, sem1)
        _finish(t0, rows0, sem0)
        _start(t0 + 2, rows0, sem0)
        _finish(t0 + 1, rows1, sem1)

    _finish(NCHUNK - 1, rows0, sem0)

    plsc.subcore_barrier()

    pltpu.sync_copy(acc.at[pl.ds(s * RPT, RPT)],
                    out_hbm.at[c].at[pl.ds(s * RPT, RPT)])

    @pl.when(s == NS - 1)
    def _out_tail():
        pltpu.sync_copy(acc.at[pl.ds(NS * RPT, TAIL)],
                        out_hbm.at[c].at[pl.ds(NS * RPT, TAIL)])


_agg = pl.kernel(
    _agg_body,
    out_type=jax.ShapeDtypeStruct((NC, N_NODES, DIM), jnp.float32),
    mesh=plsc.VectorSubcoreMesh(core_axis_name="c", subcore_axis_name="s"),
    scratch_types=[
        pltpu.VMEM_SHARED((N_NODES, DIM), jnp.float32),
        pltpu.VMEM((EPW,), jnp.int32),
        pltpu.VMEM((NCHUNK, CHUNK), jnp.int32),
        pltpu.VMEM((CHUNK, DIM), jnp.float32),
        pltpu.VMEM((CHUNK, DIM), jnp.float32),
        pltpu.SemaphoreType.DMA,
        pltpu.SemaphoreType.DMA,
    ],
)


_HI = lax.Precision.HIGHEST
_RB = 1000
_NRB = N_NODES // _RB


def _layer_math(x_blk, a_ref, w1s_ref, b1s_ref, w2_ref, b2_ref):
    h = x_blk + a_ref[0] + a_ref[1]
    t = jnp.dot(h, w1s_ref[...], precision=_HI) + b1s_ref[...]
    t = jnp.maximum(t, 0.0)
    o = jnp.dot(t, w2_ref[...], precision=_HI) + b2_ref[...]
    return jnp.maximum(o, 0.0)


def _mlp_body(x_ref, a_ref, w1s_ref, b1s_ref, w2_ref, b2_ref, o_ref):
    o_ref[...] = _layer_math(x_ref[...], a_ref, w1s_ref, b1s_ref,
                             w2_ref, b2_ref)


def _mlp(h, agg, w1s, b1s, w2, b2):
    return pl.pallas_call(
        _mlp_body,
        grid=(_NRB,),
        in_specs=[
            pl.BlockSpec((_RB, DIM), lambda i: (i, 0)),
            pl.BlockSpec((NC, _RB, DIM), lambda i: (0, i, 0)),
            pl.BlockSpec((DIM, DIM), lambda i: (0, 0)),
            pl.BlockSpec((1, DIM), lambda i: (0, 0)),
            pl.BlockSpec((DIM, DIM), lambda i: (0, 0)),
            pl.BlockSpec((1, DIM), lambda i: (0, 0)),
        ],
        out_specs=pl.BlockSpec((_RB, DIM), lambda i: (i, 0)),
        out_shape=jax.ShapeDtypeStruct((N_NODES, DIM), jnp.float32),
    )(h, agg, w1s, b1s, w2, b2)


def _head_body(x_ref, a_ref, batch_ref, w1s_ref, b1s_ref, w2_ref, b2_ref,
               l1w_ref, l1b_ref, l2w_ref, l2b_ref, o_ref, pool_acc):
    i = pl.program_id(0)
    h5 = _layer_math(x_ref[...], a_ref, w1s_ref, b1s_ref, w2_ref, b2_ref)
    b = batch_ref[0, 0, :]
    onehot = (b[:, None] == lax.broadcasted_iota(
        jnp.int32, (1, N_GRAPHS), 1)).astype(jnp.float32)
    part = lax.dot_general(onehot, h5, (((0,), (0,)), ((), ())),
                           precision=_HI)

    @pl.when(i == 0)
    def _first():
        pool_acc[...] = part

    @pl.when(i > 0)
    def _rest():
        pool_acc[...] += part

    @pl.when(i == _NRB - 1)
    def _final():
        pooled = pool_acc[...]
        u = jnp.dot(pooled, l1w_ref[...], precision=_HI) + l1b_ref[...]
        u = jnp.maximum(u, 0.0)
        o = jnp.dot(u, l2w_ref[...], precision=_HI) + l2b_ref[...]
        m = jnp.max(o, axis=-1, keepdims=True)
        e = o - m
        o_ref[...] = e - jnp.log(jnp.sum(jnp.exp(e), axis=-1, keepdims=True))


def _head(h, agg, batch_r, w1s, b1s, w2, b2, l1w, l1b, l2w, l2b):
    return pl.pallas_call(
        _head_body,
        grid=(_NRB,),
        in_specs=[
            pl.BlockSpec((_RB, DIM), lambda i: (i, 0)),
            pl.BlockSpec((NC, _RB, DIM), lambda i: (0, i, 0)),
            pl.BlockSpec((1, 1, _RB), lambda i: (i, 0, 0)),
            pl.BlockSpec((DIM, DIM), lambda i: (0, 0)),
            pl.BlockSpec((1, DIM), lambda i: (0, 0)),
            pl.BlockSpec((DIM, DIM), lambda i: (0, 0)),
            pl.BlockSpec((1, DIM), lambda i: (0, 0)),
            pl.BlockSpec((DIM, DIM), lambda i: (0, 0)),
            pl.BlockSpec((1, DIM), lambda i: (0, 0)),
            pl.BlockSpec((DIM, OUT_CH), lambda i: (0, 0)),
            pl.BlockSpec((1, OUT_CH), lambda i: (0, 0)),
        ],
        out_specs=pl.BlockSpec((N_GRAPHS, OUT_CH), lambda i: (0, 0)),
        out_shape=jax.ShapeDtypeStruct((N_GRAPHS, OUT_CH), jnp.float32),
        scratch_shapes=[pltpu.VMEM((N_GRAPHS, DIM), jnp.float32)],
    )(h, agg, batch_r, w1s, b1s, w2, b2, l1w, l1b, l2w, l2b)


def kernel(x, edge_index, batch, params):
    src = edge_index[0].astype(jnp.int32)
    dst = edge_index[1].astype(jnp.int32).reshape(NW, NCHUNK, CHUNK)
    batch_r = batch.astype(jnp.int32).reshape(_NRB, 1, _RB)

    bn_rsqrt = 1.0 / jnp.sqrt(jnp.float32(1.0 + 1e-5))
    h = x
    out = None
    for li in range(1, 6):
        p = params[f"conv{li}"]
        scale = p["g"] * bn_rsqrt
        w1s = p["W1"] * scale[None, :]
        b1s = (p["b1"] * scale + p["b"]).reshape(1, DIM)
        w2 = p["W2"]
        b2 = p["b2"].reshape(1, DIM)
        agg = _agg(h, src, dst)
        if li < 5:
            h = _mlp(h, agg, w1s, b1s, w2, b2)
        else:
            out = _head(h, agg, batch_r, w1s, b1s, w2, b2,
                        params["lin1_W"], params["lin1_b"].reshape(1, DIM),
                        params["lin2_W"], params["lin2_b"].reshape(1, OUT_CH))
    return out


# speedup vs baseline: 4.6248x; 1.0532x over previous
import functools

import jax
import jax.numpy as jnp
from jax import lax
from jax.experimental import pallas as pl
from jax.experimental.pallas import tpu as pltpu
from jax.experimental.pallas import tpu_sc as plsc

N_NODES = 10000
N_EDGES = 320000
DIM = 128
N_GRAPHS = 16
OUT_CH = 10

NC = 2
NS = 16
NW = NC * NS
CHUNK = 80
EPW = N_EDGES // NW
NCHUNK = EPW // CHUNK
RPT = 624
TAIL = N_NODES - NS * RPT

_LANES = 16


def _agg_body(h_hbm, src_hbm, dst_hbm, out_hbm,
              acc, src_v, dst_v, rows0, rows1, sem0, sem1):
    c = lax.axis_index("c")
    s = lax.axis_index("s")
    wid = c * NS + s

    src_cp = pltpu.async_copy(src_hbm.at[pl.ds(wid * EPW, EPW)], src_v, sem0)
    dst_cp = pltpu.async_copy(dst_hbm.at[wid], dst_v, sem0)

    @pl.loop(0, CHUNK)
    def _zero(r):
        for j in range(0, DIM, _LANES):
            rows0[r, pl.ds(j, _LANES)] = jnp.zeros((_LANES,), jnp.float32)

    @pl.loop(0, RPT // CHUNK)
    def _init(j):
        pltpu.sync_copy(rows0, acc.at[pl.ds(s * RPT + j * CHUNK, CHUNK)])

    pltpu.sync_copy(rows0.at[pl.ds(0, RPT % CHUNK)],
                    acc.at[pl.ds(s * RPT + RPT - RPT % CHUNK, RPT % CHUNK)])

    @pl.when(s == NS - 1)
    def _init_tail():
        pltpu.sync_copy(rows0.at[pl.ds(0, TAIL)],
                        acc.at[pl.ds(NS * RPT, TAIL)])

    src_cp.wait()
    dst_cp.wait()
    plsc.subcore_barrier()

    def _start(t, buf, sem):
        pltpu.async_copy(h_hbm.at[src_v.at[pl.ds(t * CHUNK, CHUNK)]], buf, sem)

    def _finish(t, buf, sem):
        pltpu.make_async_copy(
            h_hbm.at[src_v.at[pl.ds(t * CHUNK, CHUNK)]], buf, sem).wait()
        pltpu.sync_copy(buf, acc.at[dst_v.at[t]], add=True)

    _start(0, rows0, sem0)

    @pl.loop(0, NCHUNK // 2)
    def _chunk(k):
        t0 = 2 * k
        _start(t0 + 1, rows1, sem1)
        _finish(t0, rows0, sem0)
        _start(t0 + 2, rows0, sem0)
        _finish(t0 + 1, rows1, sem1)

    _finish(NCHUNK - 1, rows0, sem0)

    plsc.subcore_barrier()

    pltpu.sync_copy(acc.at[pl.ds(s * RPT, RPT)],
                    out_hbm.at[c].at[pl.ds(s * RPT, RPT)])

    @pl.when(s == NS - 1)
    def _out_tail():
        pltpu.sync_copy(acc.at[pl.ds(NS * RPT, TAIL)],
                        out_hbm.at[c].at[pl.ds(NS * RPT, TAIL)])


_agg = pl.kernel(
    _agg_body,
    out_type=jax.ShapeDtypeStruct((NC, N_NODES, DIM), jnp.float32),
    mesh=plsc.VectorSubcoreMesh(core_axis_name="c", subcore_axis_name="s"),
    scratch_types=[
        pltpu.VMEM_SHARED((N_NODES, DIM), jnp.float32),
        pltpu.VMEM((EPW,), jnp.int32),
        pltpu.VMEM((NCHUNK, CHUNK), jnp.int32),
        pltpu.VMEM((CHUNK, DIM), jnp.float32),
        pltpu.VMEM((CHUNK, DIM), jnp.float32),
        pltpu.SemaphoreType.DMA,
        pltpu.SemaphoreType.DMA,
    ],
)


_HI = lax.Precision.HIGHEST
_RB = 2000
_NRB = N_NODES // _RB


def _layer_math(x_blk, a0_ref, a1_ref, w1s_ref, b1s_ref, w2_ref, b2_ref):
    h = x_blk + a0_ref[...] + a1_ref[...]
    t = jnp.dot(h, w1s_ref[...], precision=_HI) + b1s_ref[...]
    t = jnp.maximum(t, 0.0)
    o = jnp.dot(t, w2_ref[...], precision=_HI) + b2_ref[...]
    return jnp.maximum(o, 0.0)


def _mlp_body(x_ref, a0_ref, a1_ref, w1s_ref, b1s_ref, w2_ref, b2_ref, o_ref):
    o_ref[...] = _layer_math(x_ref[...], a0_ref, a1_ref, w1s_ref, b1s_ref,
                             w2_ref, b2_ref)


def _mlp(h, agg, w1s, b1s, w2, b2):
    return pl.pallas_call(
        _mlp_body,
        grid=(_NRB,),
        in_specs=[
            pl.BlockSpec((_RB, DIM), lambda i: (i, 0)),
            pl.BlockSpec((_RB, DIM), lambda i: (i, 0)),
            pl.BlockSpec((_RB, DIM), lambda i: (i, 0)),
            pl.BlockSpec((DIM, DIM), lambda i: (0, 0)),
            pl.BlockSpec((1, DIM), lambda i: (0, 0)),
            pl.BlockSpec((DIM, DIM), lambda i: (0, 0)),
            pl.BlockSpec((1, DIM), lambda i: (0, 0)),
        ],
        out_specs=pl.BlockSpec((_RB, DIM), lambda i: (i, 0)),
        out_shape=jax.ShapeDtypeStruct((N_NODES, DIM), jnp.float32),
    )(h, agg[0], agg[1], w1s, b1s, w2, b2)


def _head_body(x_ref, a0_ref, a1_ref, batch_ref, w1s_ref, b1s_ref, w2_ref,
               b2_ref, l1w_ref, l1b_ref, l2w_ref, l2b_ref, o_ref, pool_acc):
    i = pl.program_id(0)
    h5 = _layer_math(x_ref[...], a0_ref, a1_ref, w1s_ref, b1s_ref, w2_ref,
                     b2_ref)
    b = batch_ref[0, 0, :]
    onehot = (b[:, None] == lax.broadcasted_iota(
        jnp.int32, (1, N_GRAPHS), 1)).astype(jnp.float32)
    part = lax.dot_general(onehot, h5, (((0,), (0,)), ((), ())),
                           precision=_HI)

    @pl.when(i == 0)
    def _first():
        pool_acc[...] = part

    @pl.when(i > 0)
    def _rest():
        pool_acc[...] += part

    @pl.when(i == _NRB - 1)
    def _final():
        pooled = pool_acc[...]
        u = jnp.dot(pooled, l1w_ref[...], precision=_HI) + l1b_ref[...]
        u = jnp.maximum(u, 0.0)
        o = jnp.dot(u, l2w_ref[...], precision=_HI) + l2b_ref[...]
        m = jnp.max(o, axis=-1, keepdims=True)
        e = o - m
        o_ref[...] = e - jnp.log(jnp.sum(jnp.exp(e), axis=-1, keepdims=True))


def _head(h, agg, batch_r, w1s, b1s, w2, b2, l1w, l1b, l2w, l2b):
    return pl.pallas_call(
        _head_body,
        grid=(_NRB,),
        in_specs=[
            pl.BlockSpec((_RB, DIM), lambda i: (i, 0)),
            pl.BlockSpec((_RB, DIM), lambda i: (i, 0)),
            pl.BlockSpec((_RB, DIM), lambda i: (i, 0)),
            pl.BlockSpec((1, 1, _RB), lambda i: (i, 0, 0)),
            pl.BlockSpec((DIM, DIM), lambda i: (0, 0)),
            pl.BlockSpec((1, DIM), lambda i: (0, 0)),
            pl.BlockSpec((DIM, DIM), lambda i: (0, 0)),
            pl.BlockSpec((1, DIM), lambda i: (0, 0)),
            pl.BlockSpec((DIM, DIM), lambda i: (0, 0)),
            pl.BlockSpec((1, DIM), lambda i: (0, 0)),
            pl.BlockSpec((DIM, OUT_CH), lambda i: (0, 0)),
            pl.BlockSpec((1, OUT_CH), lambda i: (0, 0)),
        ],
        out_specs=pl.BlockSpec((N_GRAPHS, OUT_CH), lambda i: (0, 0)),
        out_shape=jax.ShapeDtypeStruct((N_GRAPHS, OUT_CH), jnp.float32),
        scratch_shapes=[pltpu.VMEM((N_GRAPHS, DIM), jnp.float32)],
    )(h, agg[0], agg[1], batch_r, w1s, b1s, w2, b2, l1w, l1b, l2w, l2b)


def kernel(x, edge_index, batch, params):
    src = edge_index[0].astype(jnp.int32)
    dst = edge_index[1].astype(jnp.int32).reshape(NW, NCHUNK, CHUNK)
    batch_r = batch.astype(jnp.int32).reshape(_NRB, 1, _RB)

    bn_rsqrt = 1.0 / jnp.sqrt(jnp.float32(1.0 + 1e-5))
    h = x
    out = None
    for li in range(1, 6):
        p = params[f"conv{li}"]
        scale = p["g"] * bn_rsqrt
        w1s = p["W1"] * scale[None, :]
        b1s = (p["b1"] * scale + p["b"]).reshape(1, DIM)
        w2 = p["W2"]
        b2 = p["b2"].reshape(1, DIM)
        agg = _agg(h, src, dst)
        if li < 5:
            h = _mlp(h, agg, w1s, b1s, w2, b2)
        else:
            out = _head(h, agg, batch_r, w1s, b1s, w2, b2,
                        params["lin1_W"], params["lin1_b"].reshape(1, DIM),
                        params["lin2_W"], params["lin2_b"].reshape(1, OUT_CH))
    return out


# speedup vs baseline: 5.2716x; 1.1398x over previous
import functools

import jax
import jax.numpy as jnp
from jax import lax
from jax.experimental import pallas as pl
from jax.experimental.pallas import tpu as pltpu
from jax.experimental.pallas import tpu_sc as plsc

N_NODES = 10000
N_EDGES = 320000
DIM = 128
N_GRAPHS = 16
OUT_CH = 10

NC = 2
NS = 16
NW = NC * NS
CHUNK = 80
EPW = N_EDGES // NW
NCHUNK = EPW // CHUNK
RPT = 624
TAIL = N_NODES - NS * RPT

_LANES = 16


def _agg_body(h_hbm, src_hbm, dst_hbm, out_hbm,
              acc, dst_v, si0, si1, si2, rows0, rows1, rows2,
              sem0, sem1, sem2, sem3, sem4, sem5, sem6, sem7, sem8):
    c = lax.axis_index("c")
    s = lax.axis_index("s")
    wid = c * NS + s

    dst_cp = pltpu.async_copy(dst_hbm.at[wid], dst_v, sem0)

    @pl.loop(0, CHUNK)
    def _zero(r):
        for j in range(0, DIM, _LANES):
            rows0[r, pl.ds(j, _LANES)] = jnp.zeros((_LANES,), jnp.float32)

    @pl.loop(0, RPT // CHUNK)
    def _init(j):
        pltpu.sync_copy(rows0, acc.at[pl.ds(s * RPT + j * CHUNK, CHUNK)])

    pltpu.sync_copy(rows0.at[pl.ds(0, RPT % CHUNK)],
                    acc.at[pl.ds(s * RPT + RPT - RPT % CHUNK, RPT % CHUNK)])

    @pl.when(s == NS - 1)
    def _init_tail():
        pltpu.sync_copy(rows0.at[pl.ds(0, TAIL)],
                        acc.at[pl.ds(NS * RPT, TAIL)])

    dst_cp.wait()
    plsc.subcore_barrier()

    rows = (rows0, rows1, rows2)
    sidx = (si0, si1, si2)
    gsem = (sem0, sem1, sem2)
    ssem = (sem3, sem4, sem5)
    isem = (sem6, sem7, sem8)

    def _start_i(t, b):
        pltpu.async_copy(src_hbm.at[wid].at[t], sidx[b], isem[b])

    def _wait_i(t, b):
        pltpu.make_async_copy(src_hbm.at[wid].at[t], sidx[b], isem[b]).wait()

    def _start_g(t, b):
        _wait_i(t, b)
        pltpu.async_copy(h_hbm.at[sidx[b]], rows[b], gsem[b])

    def _wait_g(t, b):
        pltpu.make_async_copy(h_hbm.at[sidx[b]], rows[b], gsem[b]).wait()

    def _start_s(t, b):
        pltpu.async_copy(rows[b], acc.at[dst_v.at[t]], ssem[b], add=True)

    def _wait_s(t, b):
        pltpu.make_async_copy(rows[b], acc.at[dst_v.at[t]], ssem[b]).wait()

    _start_i(0, 0)
    _start_i(1, 1)
    _start_g(0, 0)
    _start_g(1, 1)

    @pl.loop(0, NCHUNK // 3)
    def _chunk(k):
        t = 3 * k
        for i in range(3):
            b2 = (i + 2) % 3
            n = t + i
            _start_i(n + 2, b2)
            _wait_g(n, i)
            _start_s(n, i)

            @pl.when(n >= 1)
            def _ws():
                _wait_s(n - 1, b2)

            _start_g(n + 2, b2)

    t = (NCHUNK // 3) * 3
    _wait_g(t, 0)
    _start_s(t, 0)
    _wait_g(t + 1, 1)
    _start_s(t + 1, 1)
    _wait_s(t - 1, 2)
    _wait_s(t, 0)
    _wait_s(t + 1, 1)

    plsc.subcore_barrier()

    pltpu.sync_copy(acc.at[pl.ds(s * RPT, RPT)],
                    out_hbm.at[c].at[pl.ds(s * RPT, RPT)])

    @pl.when(s == NS - 1)
    def _out_tail():
        pltpu.sync_copy(acc.at[pl.ds(NS * RPT, TAIL)],
                        out_hbm.at[c].at[pl.ds(NS * RPT, TAIL)])


_agg = pl.kernel(
    _agg_body,
    out_type=jax.ShapeDtypeStruct((NC, N_NODES, DIM), jnp.float32),
    mesh=plsc.VectorSubcoreMesh(core_axis_name="c", subcore_axis_name="s"),
    scratch_types=[
        pltpu.VMEM_SHARED((N_NODES, DIM), jnp.float32),
        pltpu.VMEM((NCHUNK, CHUNK), jnp.int32),
        pltpu.VMEM((CHUNK,), jnp.int32),
        pltpu.VMEM((CHUNK,), jnp.int32),
        pltpu.VMEM((CHUNK,), jnp.int32),
        pltpu.VMEM((CHUNK, DIM), jnp.float32),
        pltpu.VMEM((CHUNK, DIM), jnp.float32),
        pltpu.VMEM((CHUNK, DIM), jnp.float32),
    ] + [pltpu.SemaphoreType.DMA] * 9,
)


_HI = lax.Precision.HIGHEST
_RB = 2000
_NRB = N_NODES // _RB


def _layer_math(x_blk, a0_ref, a1_ref, w1s_ref, b1s_ref, w2_ref, b2_ref):
    h = x_blk + a0_ref[...] + a1_ref[...]
    t = jnp.dot(h, w1s_ref[...], precision=_HI) + b1s_ref[...]
    t = jnp.maximum(t, 0.0)
    o = jnp.dot(t, w2_ref[...], precision=_HI) + b2_ref[...]
    return jnp.maximum(o, 0.0)


def _mlp_body(x_ref, a0_ref, a1_ref, w1s_ref, b1s_ref, w2_ref, b2_ref, o_ref):
    o_ref[...] = _layer_math(x_ref[...], a0_ref, a1_ref, w1s_ref, b1s_ref,
                             w2_ref, b2_ref)


def _mlp(h, agg, w1s, b1s, w2, b2):
    return pl.pallas_call(
        _mlp_body,
        grid=(_NRB,),
        in_specs=[
            pl.BlockSpec((_RB, DIM), lambda i: (i, 0)),
            pl.BlockSpec((_RB, DIM), lambda i: (i, 0)),
            pl.BlockSpec((_RB, DIM), lambda i: (i, 0)),
            pl.BlockSpec((DIM, DIM), lambda i: (0, 0)),
            pl.BlockSpec((1, DIM), lambda i: (0, 0)),
            pl.BlockSpec((DIM, DIM), lambda i: (0, 0)),
            pl.BlockSpec((1, DIM), lambda i: (0, 0)),
        ],
        out_specs=pl.BlockSpec((_RB, DIM), lambda i: (i, 0)),
        out_shape=jax.ShapeDtypeStruct((N_NODES, DIM), jnp.float32),
    )(h, agg[0], agg[1], w1s, b1s, w2, b2)


def _head_body(x_ref, a0_ref, a1_ref, batch_ref, w1s_ref, b1s_ref, w2_ref,
               b2_ref, l1w_ref, l1b_ref, l2w_ref, l2b_ref, o_ref, pool_acc):
    i = pl.program_id(0)
    h5 = _layer_math(x_ref[...], a0_ref, a1_ref, w1s_ref, b1s_ref, w2_ref,
                     b2_ref)
    b = batch_ref[0, 0, :]
    onehot = (b[:, None] == lax.broadcasted_iota(
        jnp.int32, (1, N_GRAPHS), 1)).astype(jnp.float32)
    part = lax.dot_general(onehot, h5, (((0,), (0,)), ((), ())),
                           precision=_HI)

    @pl.when(i == 0)
    def _first():
        pool_acc[...] = part

    @pl.when(i > 0)
    def _rest():
        pool_acc[...] += part

    @pl.when(i == _NRB - 1)
    def _final():
        pooled = pool_acc[...]
        u = jnp.dot(pooled, l1w_ref[...], precision=_HI) + l1b_ref[...]
        u = jnp.maximum(u, 0.0)
        o = jnp.dot(u, l2w_ref[...], precision=_HI) + l2b_ref[...]
        m = jnp.max(o, axis=-1, keepdims=True)
        e = o - m
        o_ref[...] = e - jnp.log(jnp.sum(jnp.exp(e), axis=-1, keepdims=True))


def _head(h, agg, batch_r, w1s, b1s, w2, b2, l1w, l1b, l2w, l2b):
    return pl.pallas_call(
        _head_body,
        grid=(_NRB,),
        in_specs=[
            pl.BlockSpec((_RB, DIM), lambda i: (i, 0)),
            pl.BlockSpec((_RB, DIM), lambda i: (i, 0)),
            pl.BlockSpec((_RB, DIM), lambda i: (i, 0)),
            pl.BlockSpec((1, 1, _RB), lambda i: (i, 0, 0)),
            pl.BlockSpec((DIM, DIM), lambda i: (0, 0)),
            pl.BlockSpec((1, DIM), lambda i: (0, 0)),
            pl.BlockSpec((DIM, DIM), lambda i: (0, 0)),
            pl.BlockSpec((1, DIM), lambda i: (0, 0)),
            pl.BlockSpec((DIM, DIM), lambda i: (0, 0)),
            pl.BlockSpec((1, DIM), lambda i: (0, 0)),
            pl.BlockSpec((DIM, OUT_CH), lambda i: (0, 0)),
            pl.BlockSpec((1, OUT_CH), lambda i: (0, 0)),
        ],
        out_specs=pl.BlockSpec((N_GRAPHS, OUT_CH), lambda i: (0, 0)),
        out_shape=jax.ShapeDtypeStruct((N_GRAPHS, OUT_CH), jnp.float32),
        scratch_shapes=[pltpu.VMEM((N_GRAPHS, DIM), jnp.float32)],
    )(h, agg[0], agg[1], batch_r, w1s, b1s, w2, b2, l1w, l1b, l2w, l2b)


def kernel(x, edge_index, batch, params):
    src = edge_index[0].astype(jnp.int32).reshape(NW, NCHUNK, CHUNK)
    dst = edge_index[1].astype(jnp.int32).reshape(NW, NCHUNK, CHUNK)
    batch_r = batch.astype(jnp.int32).reshape(_NRB, 1, _RB)

    bn_rsqrt = 1.0 / jnp.sqrt(jnp.float32(1.0 + 1e-5))
    h = x
    out = None
    for li in range(1, 6):
        p = params[f"conv{li}"]
        scale = p["g"] * bn_rsqrt
        w1s = p["W1"] * scale[None, :]
        b1s = (p["b1"] * scale + p["b"]).reshape(1, DIM)
        w2 = p["W2"]
        b2 = p["b2"].reshape(1, DIM)
        agg = _agg(h, src, dst)
        if li < 5:
            h = _mlp(h, agg, w1s, b1s, w2, b2)
        else:
            out = _head(h, agg, batch_r, w1s, b1s, w2, b2,
                        params["lin1_W"], params["lin1_b"].reshape(1, DIM),
                        params["lin2_W"], params["lin2_b"].reshape(1, OUT_CH))
    return out


# speedup vs baseline: 5.5168x; 1.0465x over previous
import functools

import jax
import jax.numpy as jnp
from jax import lax
from jax.experimental import pallas as pl
from jax.experimental.pallas import tpu as pltpu
from jax.experimental.pallas import tpu_sc as plsc

N_NODES = 10000
N_EDGES = 320000
DIM = 128
N_GRAPHS = 16
OUT_CH = 10

NC = 2
NS = 16
NW = NC * NS
CHUNK = 80
EPW = N_EDGES // NW
NCHUNK = EPW // CHUNK
RPT = 624
TAIL = N_NODES - NS * RPT

_LANES = 16


def _agg_body(h_hbm, src_hbm, dst_hbm, out_hbm,
              acc, dst_v, si0, si1, si2, rows0, rows1, rows2,
              sem0, sem1, sem2, sem3, sem4, sem5, sem6, sem7, sem8):
    c = lax.axis_index("c")
    s = lax.axis_index("s")
    wid = c * NS + s

    dst_cp = pltpu.async_copy(dst_hbm.at[wid], dst_v, sem0)

    @pl.loop(0, CHUNK)
    def _zero(r):
        for j in range(0, DIM, _LANES):
            rows0[r, pl.ds(j, _LANES)] = jnp.zeros((_LANES,), jnp.float32)

    @pl.loop(0, RPT // CHUNK)
    def _init(j):
        pltpu.sync_copy(rows0, acc.at[pl.ds(s * RPT + j * CHUNK, CHUNK)])

    pltpu.sync_copy(rows0.at[pl.ds(0, RPT % CHUNK)],
                    acc.at[pl.ds(s * RPT + RPT - RPT % CHUNK, RPT % CHUNK)])

    @pl.when(s == NS - 1)
    def _init_tail():
        pltpu.sync_copy(rows0.at[pl.ds(0, TAIL)],
                        acc.at[pl.ds(NS * RPT, TAIL)])

    dst_cp.wait()
    plsc.subcore_barrier()

    rows = (rows0, rows1, rows2)
    sidx = (si0, si1, si2)
    gsem = (sem0, sem1, sem2)
    ssem = (sem3, sem4, sem5)
    isem = (sem6, sem7, sem8)

    def _start_i(t, b):
        pltpu.async_copy(src_hbm.at[wid].at[t], sidx[b], isem[b])

    def _wait_i(t, b):
        pltpu.make_async_copy(src_hbm.at[wid].at[t], sidx[b], isem[b]).wait()

    def _start_g(t, b):
        _wait_i(t, b)
        pltpu.async_copy(h_hbm.at[sidx[b]], rows[b], gsem[b])

    def _wait_g(t, b):
        pltpu.make_async_copy(h_hbm.at[sidx[b]], rows[b], gsem[b]).wait()

    def _start_s(t, b):
        pltpu.async_copy(rows[b], acc.at[dst_v.at[t]], ssem[b], add=True)

    def _wait_s(t, b):
        pltpu.make_async_copy(rows[b], acc.at[dst_v.at[t]], ssem[b]).wait()

    _start_i(0, 0)
    _start_i(1, 1)
    _start_g(0, 0)
    _start_g(1, 1)

    @pl.loop(0, NCHUNK // 3)
    def _chunk(k):
        t = 3 * k
        for i in range(3):
            b2 = (i + 2) % 3
            n = t + i
            _start_i(n + 2, b2)
            _wait_g(n, i)
            _start_s(n, i)

            @pl.when(n >= 1)
            def _ws():
                _wait_s(n - 1, b2)

            _start_g(n + 2, b2)

    t = (NCHUNK // 3) * 3
    _wait_g(t, 0)
    _start_s(t, 0)
    _wait_g(t + 1, 1)
    _start_s(t + 1, 1)
    _wait_s(t - 1, 2)
    _wait_s(t, 0)
    _wait_s(t + 1, 1)

    plsc.subcore_barrier()

    pltpu.sync_copy(acc.at[pl.ds(s * RPT, RPT)],
                    out_hbm.at[c].at[pl.ds(s * RPT, RPT)])

    @pl.when(s == NS - 1)
    def _out_tail():
        pltpu.sync_copy(acc.at[pl.ds(NS * RPT, TAIL)],
                        out_hbm.at[c].at[pl.ds(NS * RPT, TAIL)])


_agg = pl.kernel(
    _agg_body,
    out_type=jax.ShapeDtypeStruct((NC, N_NODES, DIM), jnp.float32),
    mesh=plsc.VectorSubcoreMesh(core_axis_name="c", subcore_axis_name="s"),
    scratch_types=[
        pltpu.VMEM_SHARED((N_NODES, DIM), jnp.float32),
        pltpu.VMEM((NCHUNK, CHUNK), jnp.int32),
        pltpu.VMEM((CHUNK,), jnp.int32),
        pltpu.VMEM((CHUNK,), jnp.int32),
        pltpu.VMEM((CHUNK,), jnp.int32),
        pltpu.VMEM((CHUNK, DIM), jnp.float32),
        pltpu.VMEM((CHUNK, DIM), jnp.float32),
        pltpu.VMEM((CHUNK, DIM), jnp.float32),
    ] + [pltpu.SemaphoreType.DMA] * 9,
)


_HI = lax.Precision.HIGHEST
_RB = 2000
_NRB = N_NODES // _RB


def _dot3(a, w_hi, w_lo):
    a_hi = a.astype(jnp.bfloat16)
    a_lo = (a - a_hi.astype(jnp.float32)).astype(jnp.bfloat16)
    f32 = jnp.float32
    return (jnp.dot(a_hi, w_hi, preferred_element_type=f32)
            + (jnp.dot(a_hi, w_lo, preferred_element_type=f32)
               + jnp.dot(a_lo, w_hi, preferred_element_type=f32)))


def _layer_math(x_blk, a0_ref, a1_ref, w1hi_ref, w1lo_ref, b1s_ref,
                w2hi_ref, w2lo_ref, b2_ref):
    h = x_blk + a0_ref[...] + a1_ref[...]
    t = _dot3(h, w1hi_ref[...], w1lo_ref[...]) + b1s_ref[...]
    t = jnp.maximum(t, 0.0)
    o = _dot3(t, w2hi_ref[...], w2lo_ref[...]) + b2_ref[...]
    return jnp.maximum(o, 0.0)


def _mlp_body(x_ref, a0_ref, a1_ref, w1hi_ref, w1lo_ref, b1s_ref,
              w2hi_ref, w2lo_ref, b2_ref, o_ref):
    o_ref[...] = _layer_math(x_ref[...], a0_ref, a1_ref, w1hi_ref, w1lo_ref,
                             b1s_ref, w2hi_ref, w2lo_ref, b2_ref)


def _mlp(h, agg, wsplit, b1s, b2):
    return pl.pallas_call(
        _mlp_body,
        grid=(_NRB,),
        in_specs=[
            pl.BlockSpec((_RB, DIM), lambda i: (i, 0)),
            pl.BlockSpec((_RB, DIM), lambda i: (i, 0)),
            pl.BlockSpec((_RB, DIM), lambda i: (i, 0)),
            pl.BlockSpec((DIM, DIM), lambda i: (0, 0)),
            pl.BlockSpec((DIM, DIM), lambda i: (0, 0)),
            pl.BlockSpec((1, DIM), lambda i: (0, 0)),
            pl.BlockSpec((DIM, DIM), lambda i: (0, 0)),
            pl.BlockSpec((DIM, DIM), lambda i: (0, 0)),
            pl.BlockSpec((1, DIM), lambda i: (0, 0)),
        ],
        out_specs=pl.BlockSpec((_RB, DIM), lambda i: (i, 0)),
        out_shape=jax.ShapeDtypeStruct((N_NODES, DIM), jnp.float32),
    )(h, agg[0], agg[1], wsplit[0], wsplit[1], b1s, wsplit[2], wsplit[3], b2)


def _head_body(x_ref, a0_ref, a1_ref, batch_ref, w1hi_ref, w1lo_ref,
               b1s_ref, w2hi_ref, w2lo_ref, b2_ref, l1w_ref, l1b_ref,
               l2w_ref, l2b_ref, o_ref, pool_acc):
    i = pl.program_id(0)
    h5 = _layer_math(x_ref[...], a0_ref, a1_ref, w1hi_ref, w1lo_ref,
                     b1s_ref, w2hi_ref, w2lo_ref, b2_ref)
    b = batch_ref[0, 0, :]
    onehot = (b[:, None] == lax.broadcasted_iota(
        jnp.int32, (1, N_GRAPHS), 1)).astype(jnp.float32)
    part = lax.dot_general(onehot, h5, (((0,), (0,)), ((), ())),
                           precision=_HI)

    @pl.when(i == 0)
    def _first():
        pool_acc[...] = part

    @pl.when(i > 0)
    def _rest():
        pool_acc[...] += part

    @pl.when(i == _NRB - 1)
    def _final():
        pooled = pool_acc[...]
        u = jnp.dot(pooled, l1w_ref[...], precision=_HI) + l1b_ref[...]
        u = jnp.maximum(u, 0.0)
        o = jnp.dot(u, l2w_ref[...], precision=_HI) + l2b_ref[...]
        m = jnp.max(o, axis=-1, keepdims=True)
        e = o - m
        o_ref[...] = e - jnp.log(jnp.sum(jnp.exp(e), axis=-1, keepdims=True))


def _head(h, agg, batch_r, wsplit, b1s, b2, l1w, l1b, l2w, l2b):
    return pl.pallas_call(
        _head_body,
        grid=(_NRB,),
        in_specs=[
            pl.BlockSpec((_RB, DIM), lambda i: (i, 0)),
            pl.BlockSpec((_RB, DIM), lambda i: (i, 0)),
            pl.BlockSpec((_RB, DIM), lambda i: (i, 0)),
            pl.BlockSpec((1, 1, _RB), lambda i: (i, 0, 0)),
            pl.BlockSpec((DIM, DIM), lambda i: (0, 0)),
            pl.BlockSpec((DIM, DIM), lambda i: (0, 0)),
            pl.BlockSpec((1, DIM), lambda i: (0, 0)),
            pl.BlockSpec((DIM, DIM), lambda i: (0, 0)),
            pl.BlockSpec((DIM, DIM), lambda i: (0, 0)),
            pl.BlockSpec((1, DIM), lambda i: (0, 0)),
            pl.BlockSpec((DIM, DIM), lambda i: (0, 0)),
            pl.BlockSpec((1, DIM), lambda i: (0, 0)),
            pl.BlockSpec((DIM, OUT_CH), lambda i: (0, 0)),
            pl.BlockSpec((1, OUT_CH), lambda i: (0, 0)),
        ],
        out_specs=pl.BlockSpec((N_GRAPHS, OUT_CH), lambda i: (0, 0)),
        out_shape=jax.ShapeDtypeStruct((N_GRAPHS, OUT_CH), jnp.float32),
        scratch_shapes=[pltpu.VMEM((N_GRAPHS, DIM), jnp.float32)],
    )(h, agg[0], agg[1], batch_r, wsplit[0], wsplit[1], b1s,
      wsplit[2], wsplit[3], b2, l1w, l1b, l2w, l2b)


def kernel(x, edge_index, batch, params):
    src = edge_index[0].astype(jnp.int32).reshape(NW, NCHUNK, CHUNK)
    dst = edge_index[1].astype(jnp.int32).reshape(NW, NCHUNK, CHUNK)
    batch_r = batch.astype(jnp.int32).reshape(_NRB, 1, _RB)

    bn_rsqrt = 1.0 / jnp.sqrt(jnp.float32(1.0 + 1e-5))
    h = x
    out = None
    for li in range(1, 6):
        p = params[f"conv{li}"]
        scale = p["g"] * bn_rsqrt
        w1s = p["W1"] * scale[None, :]
        b1s = (p["b1"] * scale + p["b"]).reshape(1, DIM)
        w2 = p["W2"]
        b2 = p["b2"].reshape(1, DIM)

        def _split(w):
            w_hi = w.astype(jnp.bfloat16)
            w_lo = (w - w_hi.astype(jnp.float32)).astype(jnp.bfloat16)
            return w_hi, w_lo

        w1hi, w1lo = _split(w1s)
        w2hi, w2lo = _split(w2)
        wsplit = (w1hi, w1lo, w2hi, w2lo)
        agg = _agg(h, src, dst)
        if li < 5:
            h = _mlp(h, agg, wsplit, b1s, b2)
        else:
            out = _head(h, agg, batch_r, wsplit, b1s, b2,
                        params["lin1_W"], params["lin1_b"].reshape(1, DIM),
                        params["lin2_W"], params["lin2_b"].reshape(1, OUT_CH))
    return out
